# baseline (speedup 1.0000x reference)
import functools

import jax
import jax.numpy as jnp
from jax import lax
from jax.experimental import pallas as pl
from jax.experimental.pallas import tpu as pltpu
from jax.experimental.pallas import tpu_sc as plsc

_INV_SQRT2 = 0.7071067811865476


def _make_sc_gather(B, D):
    info = plsc.get_sparse_core_info()
    NC, NS = info.num_cores, info.num_subcores
    NW = NC * NS
    b_per_w = B // NW
    mesh = plsc.VectorSubcoreMesh(core_axis_name="c", subcore_axis_name="s")

    @functools.partial(
        pl.kernel, mesh=mesh,
        out_type=jax.ShapeDtypeStruct((B, D), jnp.float32),
        scratch_types=[
            pltpu.VMEM((b_per_w,), jnp.int32),
            pltpu.VMEM((b_per_w, D), jnp.float32),
            pltpu.SemaphoreType.DMA,
        ],
    )
    def gather_rows(table_hbm, idx_hbm, out_hbm, idx_v, rows_v, sem):
        wid = lax.axis_index("s") * NC + lax.axis_index("c")
        base = wid * b_per_w
        pltpu.sync_copy(idx_hbm.at[pl.ds(base, b_per_w)], idx_v)
        pltpu.async_copy(table_hbm.at[idx_v], rows_v, sem).wait()
        pltpu.sync_copy(rows_v, out_hbm.at[pl.ds(base, b_per_w)])

    return gather_rows


def _adapter_body(cid_ref, h_ref, wd_ref, bd_ref, wu_ref, bu_ref, out_ref,
                  wd_s, wu_s, bd_s, *, bd_dim):
    n_e = wd_ref.shape[0]

    @pl.when(pl.program_id(0) == 0)
    def _build_weights():
        for e in range(n_e):
            sl = pl.ds(e * bd_dim, bd_dim)
            wd_s[:, sl] = wd_ref[e].astype(jnp.bfloat16)
            wu_s[sl, :] = wu_ref[e].astype(jnp.bfloat16)
            bd_s[0:1, sl] = bd_ref[pl.ds(e, 1), :]

    hb = h_ref[...]
    cid = jnp.reshape(cid_ref[...], (hb.shape[0], 1))
    h16 = hb.astype(jnp.bfloat16)
    z = jnp.dot(h16, wd_s[...], preferred_element_type=jnp.float32)
    z = z + bd_s[...]
    a = 0.5 * z * (1.0 + lax.erf(z * _INV_SQRT2))
    col_expert = lax.broadcasted_iota(jnp.int32, z.shape, 1) // bd_dim
    am = jnp.where(col_expert == cid, a, 0.0).astype(jnp.bfloat16)
    delta = jnp.dot(am, wu_s[...], preferred_element_type=jnp.float32)
    oh = (lax.broadcasted_iota(jnp.int32, (hb.shape[0], n_e), 1) == cid)
    bu_sel = jnp.dot(oh.astype(jnp.float32), bu_ref[...],
                     preferred_element_type=jnp.float32)
    out_ref[...] = hb + delta + bu_sel


def _tc_adapter(cs, hs, W_down, b_down, W_up, b_up):
    B, D = hs.shape
    E, _, BD = W_down.shape
    BT = 512
    return pl.pallas_call(
        functools.partial(_adapter_body, bd_dim=BD),
        grid=(B // BT,),
        in_specs=[
            pl.BlockSpec((BT,), lambda i: (i,)),
            pl.BlockSpec((BT, D), lambda i: (i, 0)),
            pl.BlockSpec((E, D, BD), lambda i: (0, 0, 0)),
            pl.BlockSpec((E, BD), lambda i: (0, 0)),
            pl.BlockSpec((E, BD, D), lambda i: (0, 0, 0)),
            pl.BlockSpec((E, D), lambda i: (0, 0)),
        ],
        out_specs=pl.BlockSpec((BT, D), lambda i: (i, 0)),
        out_shape=jax.ShapeDtypeStruct((B, D), jnp.float32),
        scratch_shapes=[
            pltpu.VMEM((D, E * BD), jnp.bfloat16),
            pltpu.VMEM((E * BD, D), jnp.bfloat16),
            pltpu.VMEM((1, E * BD), jnp.float32),
        ],
    )(cs, hs, W_down, b_down, W_up, b_up)


def kernel(h, cluster_ids, W_down, b_down, W_up, b_up):
    B, D = h.shape
    order = jnp.argsort(cluster_ids).astype(jnp.int32)
    inv = jnp.argsort(order).astype(jnp.int32)
    cs = cluster_ids[order]
    gather = _make_sc_gather(B, D)
    hs = gather(h, order)
    tmp = _tc_adapter(cs, hs, W_down, b_down, W_up, b_up)
    return gather(tmp, inv)


# speedup vs baseline: 2.3987x; 2.3987x over previous
import functools

import jax
import jax.numpy as jnp
from jax import lax
from jax.experimental import pallas as pl
from jax.experimental.pallas import tpu as pltpu

_INV_SQRT2 = 0.7071067811865476


def _adapter_body(cid_ref, h_ref, wd_ref, bd_ref, wu_ref, bu_ref, out_ref,
                  wd_s, wu_s, bd_s, am_s, *, bd_dim, d_half):
    n_e = wd_ref.shape[0]
    i = pl.program_id(0)
    j = pl.program_id(1)

    @pl.when(jnp.logical_and(i == 0, j == 0))
    def _build_weights():
        for e in range(n_e):
            sl = pl.ds(e * bd_dim, bd_dim)
            wd_s[:, sl] = wd_ref[e].astype(jnp.bfloat16)
            wu_s[sl, :] = wu_ref[e].astype(jnp.bfloat16)
            bd_s[0:1, sl] = bd_ref[pl.ds(e, 1), :]

    bt = out_ref.shape[0]
    cid = jnp.reshape(cid_ref[...], (bt, 1))

    @pl.when(j == 0)
    def _first_half_prep():
        h16 = h_ref[...].astype(jnp.bfloat16)
        z = jnp.dot(h16, wd_s[...], preferred_element_type=jnp.float32)
        z = z + bd_s[...]
        a = 0.5 * z * (1.0 + lax.erf(z * _INV_SQRT2))
        col_expert = lax.broadcasted_iota(jnp.int32, z.shape, 1) // bd_dim
        am_s[...] = jnp.where(col_expert == cid, a, 0.0).astype(jnp.bfloat16)

    dsl = pl.ds(j * d_half, d_half)
    delta = jnp.dot(am_s[...], wu_s[:, dsl], preferred_element_type=jnp.float32)
    oh = (lax.broadcasted_iota(jnp.int32, (bt, n_e), 1) == cid)
    bu_sel = jnp.dot(oh.astype(jnp.float32), bu_ref[:, dsl],
                     preferred_element_type=jnp.float32)
    out_ref[...] = h_ref[:, dsl] + delta + bu_sel


def kernel(h, cluster_ids, W_down, b_down, W_up, b_up):
    B, D = h.shape
    E, _, BD = W_down.shape
    BT = 512
    NS = 2
    DH = D // NS

    return pl.pallas_call(
        functools.partial(_adapter_body, bd_dim=BD, d_half=DH),
        grid=(B // BT, NS),
        in_specs=[
            pl.BlockSpec((BT,), lambda i, j: (i,)),
            pl.BlockSpec((BT, D), lambda i, j: (i, 0)),
            pl.BlockSpec((E, D, BD), lambda i, j: (0, 0, 0)),
            pl.BlockSpec((E, BD), lambda i, j: (0, 0)),
            pl.BlockSpec((E, BD, D), lambda i, j: (0, 0, 0)),
            pl.BlockSpec((E, D), lambda i, j: (0, 0)),
        ],
        out_specs=pl.BlockSpec((BT, DH), lambda i, j: (i, j)),
        out_shape=jax.ShapeDtypeStruct((B, D), jnp.float32),
        scratch_shapes=[
            pltpu.VMEM((D, E * BD), jnp.bfloat16),
            pltpu.VMEM((E * BD, D), jnp.bfloat16),
            pltpu.VMEM((1, E * BD), jnp.float32),
            pltpu.VMEM((BT, E * BD), jnp.bfloat16),
        ],
    )(cluster_ids, h, W_down, b_down, W_up, b_up)


# speedup vs baseline: 3.3394x; 1.3921x over previous
import functools

import jax
import jax.numpy as jnp
from jax import lax
from jax.experimental import pallas as pl
from jax.experimental.pallas import tpu as pltpu

_INV_SQRT2 = 0.7071067811865476


def _adapter_body(cid_ref, h_ref, wd_ref, bd_ref, wu_ref, bu_ref, out_ref,
                  wd_s, wu_s, bd_s, *, bd_dim):
    n_e = wd_ref.shape[0]

    @pl.when(pl.program_id(0) == 0)
    def _build_weights():
        for e in range(n_e):
            sl = pl.ds(e * bd_dim, bd_dim)
            wd_s[:, sl] = wd_ref[e].astype(jnp.bfloat16)
            wu_s[sl, :] = wu_ref[e].astype(jnp.bfloat16)
            bd_s[0:1, sl] = bd_ref[pl.ds(e, 1), :]

    hb = h_ref[...]
    cid = jnp.reshape(cid_ref[...], (hb.shape[0], 1))
    h16 = hb.astype(jnp.bfloat16)
    z = jnp.dot(h16, wd_s[...], preferred_element_type=jnp.float32)
    z = z + bd_s[...]
    a = 0.5 * z * (1.0 + lax.erf(z * _INV_SQRT2))
    col_expert = lax.broadcasted_iota(jnp.int32, z.shape, 1) // bd_dim
    am = jnp.where(col_expert == cid, a, 0.0).astype(jnp.bfloat16)
    delta = jnp.dot(am, wu_s[...], preferred_element_type=jnp.float32)
    oh = (lax.broadcasted_iota(jnp.int32, (hb.shape[0], n_e), 1) == cid)
    bu_sel = jnp.dot(oh.astype(jnp.float32), bu_ref[...],
                     preferred_element_type=jnp.float32)
    out_ref[...] = hb + delta + bu_sel


def kernel(h, cluster_ids, W_down, b_down, W_up, b_up):
    B, D = h.shape
    E, _, BD = W_down.shape
    BT = 512

    return pl.pallas_call(
        functools.partial(_adapter_body, bd_dim=BD),
        grid=(B // BT,),
        in_specs=[
            pl.BlockSpec((BT,), lambda i: (i,)),
            pl.BlockSpec((BT, D), lambda i: (i, 0)),
            pl.BlockSpec((E, D, BD), lambda i: (0, 0, 0)),
            pl.BlockSpec((E, BD), lambda i: (0, 0)),
            pl.BlockSpec((E, BD, D), lambda i: (0, 0, 0)),
            pl.BlockSpec((E, D), lambda i: (0, 0)),
        ],
        out_specs=pl.BlockSpec((BT, D), lambda i: (i, 0)),
        out_shape=jax.ShapeDtypeStruct((B, D), jnp.float32),
        scratch_shapes=[
            pltpu.VMEM((D, E * BD), jnp.bfloat16),
            pltpu.VMEM((E * BD, D), jnp.bfloat16),
            pltpu.VMEM((1, E * BD), jnp.float32),
        ],
    )(cluster_ids, h, W_down, b_down, W_up, b_up)
